# Initial kernel scaffold; baseline (speedup 1.0000x reference)
import jax
import jax.numpy as jnp
from jax.experimental import pallas as pl


def kernel(token_ids, W):
    raise NotImplementedError("write your pallas kernel here")


# speedup vs baseline: 1.8454x; 1.8454x over previous
import functools

import jax
import jax.numpy as jnp
from jax import lax
from jax.experimental import pallas as pl
from jax.experimental.pallas import tpu as pltpu
from jax.experimental.pallas import tpu_sc as plsc

_NC = 2
_NS = 16
_NW = _NC * _NS

_GC = 128
_K = 8
_BLOCK = _GC * _K


def _make_gather(B, D):
    assert B % (_NW * _BLOCK) == 0
    b_per_w = B // _NW
    n_blocks = b_per_w // _BLOCK
    mesh = plsc.VectorSubcoreMesh(core_axis_name="c", subcore_axis_name="s")

    @functools.partial(
        pl.kernel,
        out_type=jax.ShapeDtypeStruct((B, D), jnp.float32),
        mesh=mesh,
        scratch_types=[
            pltpu.VMEM((_K, _GC), jnp.int32),
            pltpu.VMEM((_BLOCK, D), jnp.float32),
            pltpu.SemaphoreType.DMA,
        ],
        compiler_params=pltpu.CompilerParams(use_tc_tiling_on_sc=False),
    )
    def gather_kernel(idx_hbm, table_hbm, out_hbm, idx_v, rows_v, sem):
        wid = lax.axis_index("s") * _NC + lax.axis_index("c")
        row0 = wid * b_per_w

        def block(g, carry):
            off = pl.multiple_of(row0 + g * _BLOCK, _BLOCK)
            pltpu.sync_copy(idx_hbm.at[pl.ds(pl.multiple_of(off // _GC, _K), _K)], idx_v)
            descs = []
            for j in range(_K):
                descs.append(
                    pltpu.async_copy(
                        table_hbm.at[idx_v.at[j]],
                        rows_v.at[pl.ds(j * _GC, _GC)],
                        sem,
                    )
                )
            for d in descs:
                d.wait()
            pltpu.sync_copy(rows_v, out_hbm.at[pl.ds(off, _BLOCK)])
            return carry

        lax.fori_loop(0, n_blocks, block, 0)

    return gather_kernel


def kernel(token_ids, W):
    B = token_ids.size
    D = W.shape[1]
    idx = token_ids.reshape(B // _GC, _GC).astype(jnp.int32)
    out = _make_gather(B, D)(idx, W)
    return out.reshape(*token_ids.shape, D)


# speedup vs baseline: 1.8497x; 1.0023x over previous
import functools

import jax
import jax.numpy as jnp
from jax import lax
from jax.experimental import pallas as pl
from jax.experimental.pallas import tpu as pltpu
from jax.experimental.pallas import tpu_sc as plsc

_NC = 2
_NS = 16
_NW = _NC * _NS

_GC = 128
_K = 4
_CHUNK = _GC * _K


def _make_gather(B, D):
    assert B % (_NW * 2 * _CHUNK) == 0
    b_per_w = B // _NW
    n_chunks = b_per_w // _CHUNK
    mesh = plsc.VectorSubcoreMesh(core_axis_name="c", subcore_axis_name="s")

    @functools.partial(
        pl.kernel,
        out_type=jax.ShapeDtypeStruct((B, D), jnp.float32),
        mesh=mesh,
        scratch_types=[
            pltpu.VMEM((2, _CHUNK), jnp.int32),
            pltpu.VMEM((2, _CHUNK, D), jnp.float32),
            pltpu.SemaphoreType.DMA,
            pltpu.SemaphoreType.DMA,
            pltpu.SemaphoreType.DMA,
            pltpu.SemaphoreType.DMA,
        ],
        compiler_params=pltpu.CompilerParams(use_tc_tiling_on_sc=False),
    )
    def gather_kernel(idx_hbm, table_hbm, out_hbm, idx_v, rows_v,
                      sg0, sg1, sw0, sw1):
        semg = (sg0, sg1)
        semw = (sw0, sw1)
        wid = lax.axis_index("s") * _NC + lax.axis_index("c")
        row0 = wid * b_per_w

        def chunk_off(c):
            return pl.multiple_of(row0 + c * _CHUNK, _CHUNK)

        def fire_gathers(c, b):
            pltpu.sync_copy(idx_hbm.at[pl.ds(chunk_off(c), _CHUNK)],
                            idx_v.at[b])
            for j in range(_K):
                pltpu.async_copy(
                    table_hbm.at[idx_v.at[b].at[pl.ds(j * _GC, _GC)]],
                    rows_v.at[b].at[pl.ds(j * _GC, _GC)],
                    semg[b],
                )

        def drain_gathers(b):
            pltpu.make_async_copy(
                table_hbm.at[pl.ds(0, _CHUNK)], rows_v.at[b], semg[b]
            ).wait()

        def fire_writeback(c, b):
            pltpu.async_copy(
                rows_v.at[b], out_hbm.at[pl.ds(chunk_off(c), _CHUNK)], semw[b]
            )

        def wait_writeback(b):
            pltpu.make_async_copy(
                rows_v.at[b], out_hbm.at[pl.ds(0, _CHUNK)], semw[b]
            ).wait()

        fire_gathers(0, 0)

        @pl.loop(0, n_chunks, step=2)
        def _(base):
            drain_gathers(0)
            fire_writeback(base, 0)

            @pl.when(base > 0)
            def _():
                wait_writeback(1)

            fire_gathers(base + 1, 1)

            drain_gathers(1)
            fire_writeback(base + 1, 1)
            wait_writeback(0)

            @pl.when(base + 2 < n_chunks)
            def _():
                fire_gathers(base + 2, 0)

        wait_writeback(1)

    return gather_kernel


def kernel(token_ids, W):
    B = token_ids.size
    D = W.shape[1]
    idx = token_ids.reshape(B).astype(jnp.int32)
    out = _make_gather(B, D)(idx, W)
    return out.reshape(*token_ids.shape, D)
